# Initial kernel scaffold; baseline (speedup 1.0000x reference)
import functools

import jax
import jax.numpy as jnp
from jax import lax
from jax.experimental import pallas as pl
from jax.experimental.pallas import tpu as pltpu
from jax.experimental.pallas import tpu_sc as plsc

D = 64
CHUNK = 128


@functools.lru_cache(maxsize=None)
def _make_sc_kernel(N, n_chunks, NC, NS):
    NW = NC * NS
    per_w = N // NW
    mesh = plsc.VectorSubcoreMesh(core_axis_name="c", subcore_axis_name="s")

    @functools.partial(
        pl.kernel,
        out_type=jax.ShapeDtypeStruct((N, D), jnp.float32),
        mesh=mesh,
        scratch_types=[
            pltpu.VMEM((n_chunks, CHUNK), jnp.int32),
            pltpu.VMEM((n_chunks, CHUNK), jnp.int32),
            pltpu.VMEM((n_chunks, CHUNK), jnp.int32),
            pltpu.VMEM((CHUNK, D), jnp.float32),
            pltpu.VMEM((CHUNK, D), jnp.float32),
            pltpu.VMEM((CHUNK, D), jnp.float32),
            pltpu.SemaphoreType.DMA,
            pltpu.SemaphoreType.DMA,
            pltpu.SemaphoreType.DMA,
        ],
    )
    def body(data_h, shift_h, total_h, wd_h, ws_h, wt_h, out_h,
             idx_d, idx_s, idx_t, acc, t1, t2, s0, s1, s2):
        wid = lax.axis_index("s") * NC + lax.axis_index("c")
        base = wid * per_w
        pltpu.sync_copy(data_h.at[wid], idx_d)
        pltpu.sync_copy(shift_h.at[wid], idx_s)
        pltpu.sync_copy(total_h.at[wid], idx_t)

        def chunk_body(c, carry):
            cp0 = pltpu.async_copy(wd_h.at[idx_d.at[c]], acc, s0)
            cp1 = pltpu.async_copy(ws_h.at[idx_s.at[c]], t1, s1)
            cp2 = pltpu.async_copy(wt_h.at[idx_t.at[c]], t2, s2)
            cp0.wait()
            cp1.wait()
            cp2.wait()

            def row_body(r, rcarry):
                for j in range(D // 16):
                    sl = pl.ds(j * 16, 16)
                    acc[r, sl] = acc[r, sl] + t1[r, sl] + t2[r, sl]
                return rcarry

            lax.fori_loop(0, CHUNK, row_body, 0)
            pltpu.sync_copy(acc, out_h.at[pl.ds(base + c * CHUNK, CHUNK)])
            return carry

        lax.fori_loop(0, n_chunks, chunk_body, 0)

    return body


def kernel(data, shift, total, W_data, W_shift, W_total):
    B, L = data.shape
    N = B * L
    info = plsc.get_sparse_core_info()
    NC, NS = info.num_cores, info.num_subcores
    NW = NC * NS
    per_w = N // NW
    n_chunks = per_w // CHUNK
    d3 = data.reshape(NW, n_chunks, CHUNK).astype(jnp.int32)
    s3 = shift.reshape(NW, n_chunks, CHUNK).astype(jnp.int32)
    t3 = total.reshape(NW, n_chunks, CHUNK).astype(jnp.int32)
    out = _make_sc_kernel(N, n_chunks, NC, NS)(
        d3, s3, t3, W_data, W_shift, W_total)
    return out.reshape(B, L, D)


# speedup vs baseline: 3.6672x; 3.6672x over previous
import functools

import jax
import jax.numpy as jnp
from jax import lax
from jax.experimental import pallas as pl
from jax.experimental.pallas import tpu as pltpu
from jax.experimental.pallas import tpu_sc as plsc

D = 64
CHUNK = 128


@functools.lru_cache(maxsize=None)
def _make_sc_kernel(N, n_chunks, NC, NS):
    NW = NC * NS
    per_w = N // NW
    mesh = plsc.VectorSubcoreMesh(core_axis_name="c", subcore_axis_name="s")

    @functools.partial(
        pl.kernel,
        out_type=jax.ShapeDtypeStruct((N, D), jnp.float32),
        mesh=mesh,
        compiler_params=pltpu.CompilerParams(use_tc_tiling_on_sc=False),
        scratch_types=[
            pltpu.VMEM((n_chunks, CHUNK), jnp.int32),
            pltpu.VMEM((n_chunks, CHUNK), jnp.int32),
            pltpu.VMEM((n_chunks, CHUNK), jnp.int32),
            pltpu.VMEM((CHUNK, D), jnp.float32),
            pltpu.VMEM((CHUNK, D), jnp.float32),
            pltpu.VMEM((CHUNK, D), jnp.float32),
            pltpu.SemaphoreType.DMA,
            pltpu.SemaphoreType.DMA,
            pltpu.SemaphoreType.DMA,
        ],
    )
    def body(data_h, shift_h, total_h, wd_h, ws_h, wt_h, out_h,
             idx_d, idx_s, idx_t, acc, t1, t2, s0, s1, s2):
        wid = lax.axis_index("s") * NC + lax.axis_index("c")
        base = wid * per_w
        pltpu.sync_copy(data_h.at[wid], idx_d)
        pltpu.sync_copy(shift_h.at[wid], idx_s)
        pltpu.sync_copy(total_h.at[wid], idx_t)

        def chunk_body(c, carry):
            cp0 = pltpu.async_copy(wd_h.at[idx_d.at[c]], acc, s0)
            cp1 = pltpu.async_copy(ws_h.at[idx_s.at[c]], t1, s1)
            cp2 = pltpu.async_copy(wt_h.at[idx_t.at[c]], t2, s2)
            cp0.wait()
            cp1.wait()
            cp2.wait()

            def row_body(r, rcarry):
                for j in range(D // 16):
                    sl = pl.ds(j * 16, 16)
                    acc[r, sl] = acc[r, sl] + t1[r, sl] + t2[r, sl]
                return rcarry

            lax.fori_loop(0, CHUNK, row_body, 0)
            pltpu.sync_copy(acc, out_h.at[pl.ds(base + c * CHUNK, CHUNK)])
            return carry

        lax.fori_loop(0, n_chunks, chunk_body, 0)

    return body


def kernel(data, shift, total, W_data, W_shift, W_total):
    B, L = data.shape
    N = B * L
    info = plsc.get_sparse_core_info()
    NC, NS = info.num_cores, info.num_subcores
    NW = NC * NS
    per_w = N // NW
    n_chunks = per_w // CHUNK
    d3 = data.reshape(NW, n_chunks, CHUNK).astype(jnp.int32)
    s3 = shift.reshape(NW, n_chunks, CHUNK).astype(jnp.int32)
    t3 = total.reshape(NW, n_chunks, CHUNK).astype(jnp.int32)
    out = _make_sc_kernel(N, n_chunks, NC, NS)(
        d3, s3, t3, W_data, W_shift, W_total)
    return out.reshape(B, L, D)


# speedup vs baseline: 3.7271x; 1.0163x over previous
import functools

import jax
import jax.numpy as jnp
from jax import lax
from jax.experimental import pallas as pl
from jax.experimental.pallas import tpu as pltpu
from jax.experimental.pallas import tpu_sc as plsc

D = 64
CHUNK = 128


@functools.lru_cache(maxsize=None)
def _make_sc_kernel(N, n_chunks, NC, NS):
    NW = NC * NS
    per_w = N // NW
    mesh = plsc.VectorSubcoreMesh(core_axis_name="c", subcore_axis_name="s")

    @functools.partial(
        pl.kernel,
        out_type=jax.ShapeDtypeStruct((N, D), jnp.float32),
        mesh=mesh,
        compiler_params=pltpu.CompilerParams(use_tc_tiling_on_sc=False),
        scratch_types=[
            pltpu.VMEM((n_chunks, CHUNK), jnp.int32),
            pltpu.VMEM((n_chunks, CHUNK), jnp.int32),
            pltpu.VMEM((n_chunks, CHUNK), jnp.int32),
            pltpu.VMEM((CHUNK, D), jnp.float32),
            pltpu.VMEM((CHUNK, D), jnp.float32),
            pltpu.VMEM((CHUNK, D), jnp.float32),
            pltpu.SemaphoreType.DMA,
            pltpu.SemaphoreType.DMA,
            pltpu.SemaphoreType.DMA,
        ],
    )
    def body(data_h, shift_h, total_h, wd_h, ws_h, wt_h, out_h,
             idx_d, idx_s, idx_t, acc, t1, t2, s0, s1, s2):
        wid = lax.axis_index("s") * NC + lax.axis_index("c")
        base = wid * per_w
        pltpu.sync_copy(data_h.at[wid], idx_d)
        pltpu.sync_copy(shift_h.at[wid], idx_s)
        pltpu.sync_copy(total_h.at[wid], idx_t)

        def chunk_body(c, carry):
            cp0 = pltpu.async_copy(wd_h.at[idx_d.at[c]], acc, s0)
            cp0.wait()
            cp1 = pltpu.async_copy(ws_h.at[idx_s.at[c]], acc, s1, add=True)
            cp2 = pltpu.async_copy(wt_h.at[idx_t.at[c]], acc, s2, add=True)
            cp1.wait()
            cp2.wait()
            pltpu.sync_copy(acc, out_h.at[pl.ds(base + c * CHUNK, CHUNK)])
            return carry

        lax.fori_loop(0, n_chunks, chunk_body, 0)

    return body


def kernel(data, shift, total, W_data, W_shift, W_total):
    B, L = data.shape
    N = B * L
    info = plsc.get_sparse_core_info()
    NC, NS = info.num_cores, info.num_subcores
    NW = NC * NS
    per_w = N // NW
    n_chunks = per_w // CHUNK
    d3 = data.reshape(NW, n_chunks, CHUNK).astype(jnp.int32)
    s3 = shift.reshape(NW, n_chunks, CHUNK).astype(jnp.int32)
    t3 = total.reshape(NW, n_chunks, CHUNK).astype(jnp.int32)
    out = _make_sc_kernel(N, n_chunks, NC, NS)(
        d3, s3, t3, W_data, W_shift, W_total)
    return out.reshape(B, L, D)


# speedup vs baseline: 4.3443x; 1.1656x over previous
import functools

import jax
import jax.numpy as jnp
from jax import lax
from jax.experimental import pallas as pl
from jax.experimental.pallas import tpu as pltpu
from jax.experimental.pallas import tpu_sc as plsc

D = 64
CHUNK = 128


@functools.lru_cache(maxsize=None)
def _make_sc_kernel(N, n_chunks, NC, NS):
    NW = NC * NS
    per_w = N // NW
    mesh = plsc.VectorSubcoreMesh(core_axis_name="c", subcore_axis_name="s")

    NBUF = 4
    n_groups = n_chunks // NBUF

    @functools.partial(
        pl.kernel,
        out_type=jax.ShapeDtypeStruct((N, D), jnp.float32),
        mesh=mesh,
        compiler_params=pltpu.CompilerParams(use_tc_tiling_on_sc=False),
        scratch_types=[
            pltpu.VMEM((n_chunks, CHUNK), jnp.int32),
            pltpu.VMEM((n_chunks, CHUNK), jnp.int32),
            pltpu.VMEM((n_chunks, CHUNK), jnp.int32),
            [pltpu.VMEM((CHUNK, D), jnp.float32)] * NBUF,
            [pltpu.SemaphoreType.DMA] * NBUF,
            [pltpu.SemaphoreType.DMA] * NBUF,
            [pltpu.SemaphoreType.DMA] * NBUF,
        ],
    )
    def body(data_h, shift_h, total_h, wd_h, ws_h, wt_h, out_h,
             idx_d, idx_s, idx_t, accs, gsems, asems, ssems):
        wid = lax.axis_index("s") * NC + lax.axis_index("c")
        base = wid * per_w
        pltpu.sync_copy(data_h.at[wid], idx_d)
        pltpu.sync_copy(shift_h.at[wid], idx_s)
        pltpu.sync_copy(total_h.at[wid], idx_t)

        def group_body(g, carry):
            for b in range(NBUF):
                c = g * NBUF + b

                @pl.when(g > 0)
                def _wait_prev_store(b=b, c=c):
                    pltpu.make_async_copy(
                        accs[b], out_h.at[pl.ds(base + (c - NBUF) * CHUNK,
                                                CHUNK)],
                        ssems[b]).wait()

                pltpu.async_copy(wd_h.at[idx_d.at[c]], accs[b], gsems[b])
            for b in range(NBUF):
                c = g * NBUF + b
                pltpu.make_async_copy(wd_h.at[idx_d.at[c]], accs[b],
                                      gsems[b]).wait()
                pltpu.async_copy(ws_h.at[idx_s.at[c]], accs[b], asems[b],
                                 add=True)
                pltpu.async_copy(wt_h.at[idx_t.at[c]], accs[b], asems[b],
                                 add=True)
            for b in range(NBUF):
                c = g * NBUF + b
                add_cp = pltpu.make_async_copy(ws_h.at[idx_s.at[c]], accs[b],
                                               asems[b])
                add_cp.wait()
                add_cp.wait()
                pltpu.async_copy(accs[b],
                                 out_h.at[pl.ds(base + c * CHUNK, CHUNK)],
                                 ssems[b])
            return carry

        lax.fori_loop(0, n_groups, group_body, 0)
        for b in range(NBUF):
            c = (n_groups - 1) * NBUF + b
            pltpu.make_async_copy(
                accs[b], out_h.at[pl.ds(base + c * CHUNK, CHUNK)],
                ssems[b]).wait()

    return body


def kernel(data, shift, total, W_data, W_shift, W_total):
    B, L = data.shape
    N = B * L
    info = plsc.get_sparse_core_info()
    NC, NS = info.num_cores, info.num_subcores
    NW = NC * NS
    per_w = N // NW
    n_chunks = per_w // CHUNK
    d3 = data.reshape(NW, n_chunks, CHUNK).astype(jnp.int32)
    s3 = shift.reshape(NW, n_chunks, CHUNK).astype(jnp.int32)
    t3 = total.reshape(NW, n_chunks, CHUNK).astype(jnp.int32)
    out = _make_sc_kernel(N, n_chunks, NC, NS)(
        d3, s3, t3, W_data, W_shift, W_total)
    return out.reshape(B, L, D)
